# baseline (speedup 1.0000x reference)
import functools

import jax
import jax.numpy as jnp
from jax import lax
from jax.experimental import pallas as pl
from jax.experimental.pallas import tpu as pltpu
from jax.experimental.pallas import tpu_sc as plsc

N = 10000
E = 320000
D = 128

NC = 2
NS = 16
NW = NC * NS

EPW = E // NW
CH = 40
NCHUNK = EPW // CH
NBUF = 4
IBLK = 25
NIB = NCHUNK // IBLK
NZC = N // CH


def _sc_aggregate_body(h_hbm, eidx_hbm, out_hbm, ibuf, rows, acc,
                       sem_z, sem_x, sem_g):
    cid = lax.axis_index("c")
    sid = lax.axis_index("s")
    wid = cid * NS + sid

    pltpu.async_copy(eidx_hbm.at[wid, 0], ibuf.at[0], sem_x.at[0])
    pltpu.async_copy(eidx_hbm.at[wid, 1], ibuf.at[1], sem_x.at[1])

    pltpu.make_async_copy(eidx_hbm.at[wid, 0], ibuf.at[0], sem_x.at[0]).wait()
    for j in range(1, NBUF):
        pltpu.async_copy(h_hbm.at[ibuf.at[0, j, 0]], rows.at[j], sem_g.at[j])

    def _zero_row(i):
        for j in range(D // 16):
            rows[0, i, pl.ds(j * 16, 16)] = jnp.zeros((16,), jnp.float32)
    pl.loop(0, CH)(_zero_row)
    for j in range(NZC // NS):
        c = sid + NS * j
        pltpu.async_copy(rows.at[0], acc.at[pl.ds(c * CH, CH), :], sem_z)

    @pl.when(sid < NZC % NS)
    def _():
        pltpu.async_copy(rows.at[0],
                         acc.at[pl.ds((sid + NS * (NZC // NS)) * CH, CH), :],
                         sem_z)
    for j in range(NZC // NS):
        pltpu.make_async_copy(rows.at[0], acc.at[pl.ds(0, CH), :],
                              sem_z).wait()

    @pl.when(sid < NZC % NS)
    def _():
        pltpu.make_async_copy(rows.at[0], acc.at[pl.ds(0, CH), :],
                              sem_z).wait()
    plsc.subcore_barrier()

    pltpu.async_copy(h_hbm.at[ibuf.at[0, 0, 0]], rows.at[0], sem_g.at[0])

    def _idx_wait(buf):
        pltpu.make_async_copy(eidx_hbm.at[wid, 0], ibuf.at[buf],
                              sem_x.at[buf]).wait()

    def _block(bB, jb, kbase, last):
        buf = jb % 2
        for j in range(IBLK):
            k = kbase + j
            b = (jb + j) % NBUF
            pltpu.make_async_copy(h_hbm.at[pl.ds(0, CH)], rows.at[b],
                                  sem_g.at[b]).wait()
            pltpu.sync_copy(rows.at[b], acc.at[ibuf.at[buf, j, 1]], add=True)
            if j == IBLK - NBUF and not last:
                _idx_wait(1 - buf)
            if not (last and j >= IBLK - NBUF):
                buf2, j2 = (buf, j + NBUF) if j < IBLK - NBUF else \
                           (1 - buf, j + NBUF - IBLK)
                pltpu.async_copy(h_hbm.at[ibuf.at[buf2, j2, 0]], rows.at[b],
                                 sem_g.at[b])

    def _quad(g):
        for jb in range(4):
            bB = 4 * g + jb
            _block(bB, jb, (4 * g + jb) * IBLK, False)
            pltpu.async_copy(eidx_hbm.at[wid, bB + 2], ibuf.at[jb % 2],
                             sem_x.at[jb % 2])
    pl.loop(0, 2)(_quad)

    for bB in (8, 9):
        _block(bB, bB % 4, bB * IBLK, bB == NIB - 1)

    plsc.subcore_barrier()

    for j in range(NZC // NS):
        c = (sid + NS * j) * CH
        pltpu.async_copy(acc.at[pl.ds(c, CH), :],
                         out_hbm.at[cid, pl.ds(c, CH), :], sem_z)

    @pl.when(sid < NZC % NS)
    def _():
        c = (sid + NS * (NZC // NS)) * CH
        pltpu.async_copy(acc.at[pl.ds(c, CH), :],
                         out_hbm.at[cid, pl.ds(c, CH), :], sem_z)
    for j in range(NZC // NS):
        pltpu.make_async_copy(acc.at[pl.ds(0, CH), :],
                              out_hbm.at[cid, pl.ds(0, CH), :], sem_z).wait()

    @pl.when(sid < NZC % NS)
    def _():
        pltpu.make_async_copy(acc.at[pl.ds(0, CH), :],
                              out_hbm.at[cid, pl.ds(0, CH), :], sem_z).wait()


_sc_aggregate = pl.kernel(
    _sc_aggregate_body,
    out_type=jax.ShapeDtypeStruct((NC, N, D), jnp.float32),
    mesh=plsc.VectorSubcoreMesh(core_axis_name="c", subcore_axis_name="s"),
    scratch_types=[
        pltpu.VMEM((2, IBLK, 2, CH), jnp.int32),
        pltpu.VMEM((NBUF, CH, D), jnp.float32),
        pltpu.VMEM_SHARED((N, D), jnp.float32),
        pltpu.SemaphoreType.DMA,
        pltpu.SemaphoreType.DMA((2,)),
        pltpu.SemaphoreType.DMA((NBUF,)),
    ],
)


RB = 2000


def _root_body(h_ref, wo_ref, br_ref, o_ref):
    o_ref[...] = (jnp.dot(h_ref[...], wo_ref[...],
                          preferred_element_type=jnp.float32) + br_ref[...])


def _root(h, wo, br):
    return pl.pallas_call(
        _root_body,
        grid=(N // RB,),
        in_specs=[
            pl.BlockSpec((RB, D), lambda i: (i, 0)),
            pl.BlockSpec((D, D), lambda i: (0, 0)),
            pl.BlockSpec((D,), lambda i: (0,)),
        ],
        out_specs=pl.BlockSpec((RB, D), lambda i: (i, 0)),
        out_shape=jax.ShapeDtypeStruct((N, D), jnp.float32),
    )(h, wo, br)


def _comb_body(do_relu, p_ref, r_ref, wr_ref, o_ref):
    agg = p_ref[0] + p_ref[1]
    o = (jnp.dot(agg, wr_ref[...], preferred_element_type=jnp.float32)
         + r_ref[...])
    if do_relu:
        o = jnp.maximum(o, 0.0)
    o_ref[...] = o


def _comb(p, r, wr, do_relu):
    return pl.pallas_call(
        functools.partial(_comb_body, do_relu),
        grid=(N // RB,),
        in_specs=[
            pl.BlockSpec((NC, RB, D), lambda i: (0, i, 0)),
            pl.BlockSpec((RB, D), lambda i: (i, 0)),
            pl.BlockSpec((D, D), lambda i: (0, 0)),
        ],
        out_specs=pl.BlockSpec((RB, D), lambda i: (i, 0)),
        out_shape=jax.ShapeDtypeStruct((N, D), jnp.float32),
    )(p, r, wr)


def kernel(x, edge_index, Wr0, br0, Wo0, Wr1, br1, Wo1, Wr2, br2, Wo2):
    eidx = (edge_index.reshape(2, NW, NCHUNK, CH).transpose(1, 2, 0, 3)
            .reshape(NW, NIB, IBLK, 2, CH))
    h = x
    for i, (wr, br, wo) in enumerate(
            ((Wr0, br0, Wo0), (Wr1, br1, Wo1), (Wr2, br2, Wo2))):
        r = _root(h, wo, br)
        p = _sc_aggregate(h, eidx)
        h = _comb(p, r, wr, do_relu=(i < 2))
    return h


# speedup vs baseline: 1.0279x; 1.0279x over previous
import functools

import jax
import jax.numpy as jnp
from jax import lax
from jax.experimental import pallas as pl
from jax.experimental.pallas import tpu as pltpu
from jax.experimental.pallas import tpu_sc as plsc

N = 10000
E = 320000
D = 128

NC = 2
NS = 16
NW = NC * NS

EPW = E // NW
CH = 40
NCHUNK = EPW // CH
NBUF = 4
IBLK = 10
NIB = NCHUNK // IBLK
NZC = N // CH


def _sc_aggregate_body(h_hbm, eidx_hbm, out_hbm, ibuf, rows, acc,
                       sem_z, sem_x, sem_g):
    cid = lax.axis_index("c")
    sid = lax.axis_index("s")
    wid = cid * NS + sid

    pltpu.async_copy(eidx_hbm.at[wid, 0], ibuf.at[0], sem_x.at[0])
    pltpu.async_copy(eidx_hbm.at[wid, 1], ibuf.at[1], sem_x.at[1])

    pltpu.make_async_copy(eidx_hbm.at[wid, 0], ibuf.at[0], sem_x.at[0]).wait()
    for j in range(1, NBUF):
        pltpu.async_copy(h_hbm.at[ibuf.at[0, j, 0]], rows.at[j], sem_g.at[j])

    def _zero_row(i):
        for j in range(D // 16):
            rows[0, i, pl.ds(j * 16, 16)] = jnp.zeros((16,), jnp.float32)
    pl.loop(0, CH)(_zero_row)
    for j in range(NZC // NS):
        c = sid + NS * j
        pltpu.async_copy(rows.at[0], acc.at[pl.ds(c * CH, CH), :], sem_z)

    @pl.when(sid < NZC % NS)
    def _():
        pltpu.async_copy(rows.at[0],
                         acc.at[pl.ds((sid + NS * (NZC // NS)) * CH, CH), :],
                         sem_z)
    for j in range(NZC // NS):
        pltpu.make_async_copy(rows.at[0], acc.at[pl.ds(0, CH), :],
                              sem_z).wait()

    @pl.when(sid < NZC % NS)
    def _():
        pltpu.make_async_copy(rows.at[0], acc.at[pl.ds(0, CH), :],
                              sem_z).wait()
    plsc.subcore_barrier()

    pltpu.async_copy(h_hbm.at[ibuf.at[0, 0, 0]], rows.at[0], sem_g.at[0])

    def _idx_wait(buf):
        pltpu.make_async_copy(eidx_hbm.at[wid, 0], ibuf.at[buf],
                              sem_x.at[buf]).wait()

    def _block(jb, kbase, buf, last):
        for j in range(IBLK):
            k = kbase + j
            b = (2 * jb + j) % NBUF
            pltpu.make_async_copy(h_hbm.at[pl.ds(0, CH)], rows.at[b],
                                  sem_g.at[b]).wait()
            pltpu.sync_copy(rows.at[b], acc.at[ibuf.at[buf, j, 1]], add=True)
            if j == IBLK - NBUF and not last:
                _idx_wait(1 - buf)
            if not (last and j >= IBLK - NBUF):
                buf2, j2 = (buf, j + NBUF) if j < IBLK - NBUF else \
                           (1 - buf, j + NBUF - IBLK)
                pltpu.async_copy(h_hbm.at[ibuf.at[buf2, j2, 0]], rows.at[b],
                                 sem_g.at[b])

    def _pair(g):
        for jb in range(2):
            bB = 2 * g + jb
            _block(jb, bB * IBLK, jb % 2, False)
            if jb == 0:
                pltpu.async_copy(eidx_hbm.at[wid, bB + 2], ibuf.at[0],
                                 sem_x.at[0])
            else:
                @pl.when(bB + 2 < NIB)
                def _():
                    pltpu.async_copy(eidx_hbm.at[wid, bB + 2], ibuf.at[1],
                                     sem_x.at[1])
    pl.loop(0, (NIB - 1) // 2)(_pair)

    _block(0, (NIB - 1) * IBLK, (NIB - 1) % 2, True)

    plsc.subcore_barrier()

    for j in range(NZC // NS):
        c = (sid + NS * j) * CH
        pltpu.async_copy(acc.at[pl.ds(c, CH), :],
                         out_hbm.at[cid, pl.ds(c, CH), :], sem_z)

    @pl.when(sid < NZC % NS)
    def _():
        c = (sid + NS * (NZC // NS)) * CH
        pltpu.async_copy(acc.at[pl.ds(c, CH), :],
                         out_hbm.at[cid, pl.ds(c, CH), :], sem_z)
    for j in range(NZC // NS):
        pltpu.make_async_copy(acc.at[pl.ds(0, CH), :],
                              out_hbm.at[cid, pl.ds(0, CH), :], sem_z).wait()

    @pl.when(sid < NZC % NS)
    def _():
        pltpu.make_async_copy(acc.at[pl.ds(0, CH), :],
                              out_hbm.at[cid, pl.ds(0, CH), :], sem_z).wait()


_sc_aggregate = pl.kernel(
    _sc_aggregate_body,
    out_type=jax.ShapeDtypeStruct((NC, N, D), jnp.float32),
    mesh=plsc.VectorSubcoreMesh(core_axis_name="c", subcore_axis_name="s"),
    scratch_types=[
        pltpu.VMEM((2, IBLK, 2, CH), jnp.int32),
        pltpu.VMEM((NBUF, CH, D), jnp.float32),
        pltpu.VMEM_SHARED((N, D), jnp.float32),
        pltpu.SemaphoreType.DMA,
        pltpu.SemaphoreType.DMA((2,)),
        pltpu.SemaphoreType.DMA((NBUF,)),
    ],
)


RB = 2000


def _dense_body(do_relu, p_ref, h_ref, wr_ref, br_ref, wo_ref, o_ref):
    agg = p_ref[0] + p_ref[1]
    o = (jnp.dot(agg, wr_ref[...], preferred_element_type=jnp.float32)
         + br_ref[...]
         + jnp.dot(h_ref[...], wo_ref[...], preferred_element_type=jnp.float32))
    if do_relu:
        o = jnp.maximum(o, 0.0)
    o_ref[...] = o


def _dense(p, h, wr, br, wo, do_relu):
    return pl.pallas_call(
        functools.partial(_dense_body, do_relu),
        grid=(N // RB,),
        in_specs=[
            pl.BlockSpec((NC, RB, D), lambda i: (0, i, 0)),
            pl.BlockSpec((RB, D), lambda i: (i, 0)),
            pl.BlockSpec((D, D), lambda i: (0, 0)),
            pl.BlockSpec((D,), lambda i: (0,)),
            pl.BlockSpec((D, D), lambda i: (0, 0)),
        ],
        out_specs=pl.BlockSpec((RB, D), lambda i: (i, 0)),
        out_shape=jax.ShapeDtypeStruct((N, D), jnp.float32),
    )(p, h, wr, br, wo)


def kernel(x, edge_index, Wr0, br0, Wo0, Wr1, br1, Wo1, Wr2, br2, Wo2):
    eidx = (edge_index.reshape(2, NW, NCHUNK, CH).transpose(1, 2, 0, 3)
            .reshape(NW, NIB, IBLK, 2, CH))
    h = x
    for i, (wr, br, wo) in enumerate(
            ((Wr0, br0, Wo0), (Wr1, br1, Wo1), (Wr2, br2, Wo2))):
        p = _sc_aggregate(h, eidx)
        h = _dense(p, h, wr, br, wo, do_relu=(i < 2))
    return h


# speedup vs baseline: 1.0379x; 1.0098x over previous
import functools

import jax
import jax.numpy as jnp
from jax import lax
from jax.experimental import pallas as pl
from jax.experimental.pallas import tpu as pltpu
from jax.experimental.pallas import tpu_sc as plsc

N = 10000
E = 320000
D = 128

NC = 2
NS = 16
NW = NC * NS

EPW = E // NW
CH = 40
NCHUNK = EPW // CH
NBUF = 4
IBLK = 10
NIB = NCHUNK // IBLK
NZC = N // CH


def _sc_aggregate_body(h_hbm, eidx_hbm, out_hbm, ibuf, rows, acc,
                       sem_z, sem_x, sem_g):
    cid = lax.axis_index("c")
    sid = lax.axis_index("s")
    wid = cid * NS + sid

    pltpu.async_copy(eidx_hbm.at[wid, 0], ibuf.at[0], sem_x.at[0])
    pltpu.async_copy(eidx_hbm.at[wid, 1], ibuf.at[1], sem_x.at[1])

    pltpu.make_async_copy(eidx_hbm.at[wid, 0], ibuf.at[0], sem_x.at[0]).wait()
    for j in range(1, NBUF):
        pltpu.async_copy(h_hbm.at[ibuf.at[0, j, 0]], rows.at[j], sem_g.at[j])

    def _zero_row(i):
        for j in range(D // 16):
            rows[0, i, pl.ds(j * 16, 16)] = jnp.zeros((16,), jnp.float32)
    pl.loop(0, CH)(_zero_row)
    for j in range(NZC // NS):
        c = sid + NS * j
        pltpu.async_copy(rows.at[0], acc.at[pl.ds(c * CH, CH), :], sem_z)

    @pl.when(sid < NZC % NS)
    def _():
        pltpu.async_copy(rows.at[0],
                         acc.at[pl.ds((sid + NS * (NZC // NS)) * CH, CH), :],
                         sem_z)
    for j in range(NZC // NS):
        pltpu.make_async_copy(rows.at[0], acc.at[pl.ds(0, CH), :],
                              sem_z).wait()

    @pl.when(sid < NZC % NS)
    def _():
        pltpu.make_async_copy(rows.at[0], acc.at[pl.ds(0, CH), :],
                              sem_z).wait()
    plsc.subcore_barrier()

    pltpu.async_copy(h_hbm.at[ibuf.at[0, 0, 0]], rows.at[0], sem_g.at[0])

    def _idx_wait(buf):
        pltpu.make_async_copy(eidx_hbm.at[wid, 0], ibuf.at[buf],
                              sem_x.at[buf]).wait()

    def _block(jb, kbase, buf, last):
        for j in range(IBLK):
            k = kbase + j
            b = (2 * jb + j) % NBUF
            pltpu.make_async_copy(h_hbm.at[pl.ds(0, CH)], rows.at[b],
                                  sem_g.at[b]).wait()
            pltpu.sync_copy(rows.at[b], acc.at[ibuf.at[buf, j, 1]], add=True)
            if j == IBLK - NBUF and not last:
                _idx_wait(1 - buf)
            if not (last and j >= IBLK - NBUF):
                buf2, j2 = (buf, j + NBUF) if j < IBLK - NBUF else \
                           (1 - buf, j + NBUF - IBLK)
                pltpu.async_copy(h_hbm.at[ibuf.at[buf2, j2, 0]], rows.at[b],
                                 sem_g.at[b])

    def _pair(g):
        for jb in range(2):
            bB = 2 * g + jb
            _block(jb, bB * IBLK, jb % 2, False)
            if jb == 0:
                pltpu.async_copy(eidx_hbm.at[wid, bB + 2], ibuf.at[0],
                                 sem_x.at[0])
            else:
                @pl.when(bB + 2 < NIB)
                def _():
                    pltpu.async_copy(eidx_hbm.at[wid, bB + 2], ibuf.at[1],
                                     sem_x.at[1])
    pl.loop(0, (NIB - 1) // 2)(_pair)

    _block(0, (NIB - 1) * IBLK, (NIB - 1) % 2, True)

    plsc.subcore_barrier()

    for j in range(NZC // NS):
        c = (sid + NS * j) * CH
        pltpu.async_copy(acc.at[pl.ds(c, CH), :],
                         out_hbm.at[cid, pl.ds(c, CH), :], sem_z)

    @pl.when(sid < NZC % NS)
    def _():
        c = (sid + NS * (NZC // NS)) * CH
        pltpu.async_copy(acc.at[pl.ds(c, CH), :],
                         out_hbm.at[cid, pl.ds(c, CH), :], sem_z)
    for j in range(NZC // NS):
        pltpu.make_async_copy(acc.at[pl.ds(0, CH), :],
                              out_hbm.at[cid, pl.ds(0, CH), :], sem_z).wait()

    @pl.when(sid < NZC % NS)
    def _():
        pltpu.make_async_copy(acc.at[pl.ds(0, CH), :],
                              out_hbm.at[cid, pl.ds(0, CH), :], sem_z).wait()


_sc_aggregate = pl.kernel(
    _sc_aggregate_body,
    out_type=jax.ShapeDtypeStruct((NC, N, D), jnp.float32),
    mesh=plsc.VectorSubcoreMesh(core_axis_name="c", subcore_axis_name="s"),
    scratch_types=[
        pltpu.VMEM((2, IBLK, 2, CH), jnp.int32),
        pltpu.VMEM((NBUF, CH, D), jnp.float32),
        pltpu.VMEM_SHARED((N, D), jnp.float32),
        pltpu.SemaphoreType.DMA,
        pltpu.SemaphoreType.DMA((2,)),
        pltpu.SemaphoreType.DMA((NBUF,)),
    ],
)


RB = 5000


def _dense_body(do_relu, p_ref, h_ref, wr_ref, br_ref, wo_ref, o_ref):
    agg = p_ref[0] + p_ref[1]
    o = (jnp.dot(agg, wr_ref[...], preferred_element_type=jnp.float32)
         + br_ref[...]
         + jnp.dot(h_ref[...], wo_ref[...], preferred_element_type=jnp.float32))
    if do_relu:
        o = jnp.maximum(o, 0.0)
    o_ref[...] = o


def _dense(p, h, wr, br, wo, do_relu):
    return pl.pallas_call(
        functools.partial(_dense_body, do_relu),
        grid=(N // RB,),
        in_specs=[
            pl.BlockSpec((NC, RB, D), lambda i: (0, i, 0)),
            pl.BlockSpec((RB, D), lambda i: (i, 0)),
            pl.BlockSpec((D, D), lambda i: (0, 0)),
            pl.BlockSpec((D,), lambda i: (0,)),
            pl.BlockSpec((D, D), lambda i: (0, 0)),
        ],
        out_specs=pl.BlockSpec((RB, D), lambda i: (i, 0)),
        out_shape=jax.ShapeDtypeStruct((N, D), jnp.float32),
    )(p, h, wr, br, wo)


def kernel(x, edge_index, Wr0, br0, Wo0, Wr1, br1, Wo1, Wr2, br2, Wo2):
    eidx = (edge_index.reshape(2, NW, NCHUNK, CH).transpose(1, 2, 0, 3)
            .reshape(NW, NIB, IBLK, 2, CH))
    h = x
    for i, (wr, br, wo) in enumerate(
            ((Wr0, br0, Wo0), (Wr1, br1, Wo1), (Wr2, br2, Wo2))):
        p = _sc_aggregate(h, eidx)
        h = _dense(p, h, wr, br, wo, do_relu=(i < 2))
    return h
